# baseline (speedup 1.0000x reference)
import jax
import jax.numpy as jnp
from jax.experimental import pallas as pl
from jax.experimental.pallas import tpu as pltpu

_NCHUNK = 8


def _dma_body(x_ref, o_ref, sems):
    rows = x_ref.shape[0]
    step = rows // _NCHUNK
    copies = [
        pltpu.make_async_copy(
            x_ref.at[pl.ds(i * step, step)],
            o_ref.at[pl.ds(i * step, step)],
            sems.at[i],
        )
        for i in range(_NCHUNK)
    ]
    for c in copies:
        c.start()
    for c in copies:
        c.wait()


def kernel(inputs, medians):
    del medians
    B, S, D = inputs.shape
    rows = B * S
    x = inputs.reshape(rows, D)
    out = pl.pallas_call(
        _dma_body,
        in_specs=[pl.BlockSpec(memory_space=pl.ANY)],
        out_specs=pl.BlockSpec(memory_space=pl.ANY),
        out_shape=jax.ShapeDtypeStruct((rows, D), inputs.dtype),
        scratch_shapes=[pltpu.SemaphoreType.DMA((_NCHUNK,))],
    )(x)
    return out.reshape(B, S, D)


# speedup vs baseline: 49.0471x; 49.0471x over previous
import jax
import jax.numpy as jnp
from jax.experimental import pallas as pl
from jax.experimental.pallas import tpu as pltpu

_ROWS_PER_BLOCK = 1024


def _copy_body(x_ref, o_ref):
    o_ref[...] = x_ref[...]


def kernel(inputs, medians):
    del medians
    B, S, D = inputs.shape
    rows = B * S
    x = inputs.reshape(rows, D)
    R = _ROWS_PER_BLOCK
    out = pl.pallas_call(
        _copy_body,
        grid=(rows // R,),
        in_specs=[pl.BlockSpec((R, D), lambda i: (i, 0))],
        out_specs=pl.BlockSpec((R, D), lambda i: (i, 0)),
        out_shape=jax.ShapeDtypeStruct((rows, D), inputs.dtype),
    )(x)
    return out.reshape(B, S, D)
